# Initial kernel scaffold; baseline (speedup 1.0000x reference)
import functools

import jax
import jax.numpy as jnp
from jax import lax
from jax.experimental import pallas as pl
from jax.experimental.pallas import tpu as pltpu
from jax.experimental.pallas import tpu_sc as plsc

N = 10000
D = 128
E = 320000

NC = 2
NS = 16
NW = NC * NS

CHUNK = 128
EDGES_PER_TILE = -(-E // (NW * CHUNK)) * CHUNK
EPAD = EDGES_PER_TILE * NW
NCHUNK = EDGES_PER_TILE // CHUNK
ROWS_PER_TILE = -(-N // (NS * CHUNK)) * CHUNK
NPAD = ROWS_PER_TILE * NS


def _sc_body(src_hbm, dst_hbm, attr_hbm, x_hbm, out_hbm,
             src_v, dst_v, attr_v, rows_v, acc, sem):
    c = lax.axis_index("c")
    s = lax.axis_index("s")
    w = c * NS + s

    @pl.loop(0, CHUNK)
    def _zero_rows(i):
        for j in range(D // 16):
            rows_v[i, pl.ds(j * 16, 16)] = jnp.zeros((16,), jnp.float32)

    for j in range(ROWS_PER_TILE // CHUNK):
        pltpu.sync_copy(rows_v, acc.at[pl.ds(s * ROWS_PER_TILE + j * CHUNK, CHUNK)])
    plsc.subcore_barrier()

    base0 = w * EDGES_PER_TILE

    @pl.loop(0, NCHUNK)
    def _edge_chunk(g):
        base = base0 + g * CHUNK
        pltpu.sync_copy(src_hbm.at[pl.ds(base, CHUNK)], src_v)
        pltpu.sync_copy(dst_hbm.at[pl.ds(base, CHUNK)], dst_v)
        pltpu.sync_copy(attr_hbm.at[pl.ds(base, CHUNK)], attr_v)
        pltpu.async_copy(x_hbm.at[src_v], rows_v, sem).wait()

        @pl.loop(0, CHUNK)
        def _scale(e):
            a = plsc.load_gather(attr_v, [jnp.full((16,), e, jnp.int32)])
            for j in range(D // 16):
                sl = pl.ds(j * 16, 16)
                rows_v[e, sl] = rows_v[e, sl] * a

        pltpu.sync_copy(rows_v, acc.at[dst_v], add=True)

    plsc.subcore_barrier()

    obase = c * NPAD + s * ROWS_PER_TILE
    for j in range(ROWS_PER_TILE // CHUNK):
        pltpu.sync_copy(acc.at[pl.ds(s * ROWS_PER_TILE + j * CHUNK, CHUNK)], rows_v)
        pltpu.sync_copy(rows_v, out_hbm.at[pl.ds(obase + j * CHUNK, CHUNK)])


_sc_agg = pl.kernel(
    _sc_body,
    out_type=jax.ShapeDtypeStruct((NC * NPAD, D), jnp.float32),
    mesh=plsc.VectorSubcoreMesh(core_axis_name="c", subcore_axis_name="s"),
    scratch_types=[
        pltpu.VMEM((CHUNK,), jnp.int32),
        pltpu.VMEM((CHUNK,), jnp.int32),
        pltpu.VMEM((CHUNK,), jnp.float32),
        pltpu.VMEM((CHUNK, D), jnp.float32),
        pltpu.VMEM_SHARED((NPAD, D), jnp.float32),
        pltpu.SemaphoreType.DMA,
    ],
)


def _tc_body(part_ref, w_ref, b_ref, gamma_ref, beta_ref, out_ref):
    agg = part_ref[0:N, :] + part_ref[NPAD:NPAD + N, :]
    y = jnp.dot(agg, w_ref[...], preferred_element_type=jnp.float32)
    y = y + b_ref[...]
    mean = jnp.mean(y, axis=0, keepdims=True)
    yc = y - mean
    var = jnp.mean(yc * yc, axis=0, keepdims=True)
    scale = lax.rsqrt(var + 1e-5) * gamma_ref[...]
    out_ref[...] = jnp.maximum(yc * scale + beta_ref[...], 0.0)


@jax.jit
def _run(x, src, dst, attr, W, b, gamma, beta):
    pad = EPAD - E
    src_p = jnp.pad(src, (0, pad))
    dst_p = jnp.pad(dst, (0, pad))
    attr_p = jnp.pad(attr, (0, pad))

    partial = _sc_agg(src_p, dst_p, attr_p, x)

    out = pl.pallas_call(
        _tc_body,
        out_shape=jax.ShapeDtypeStruct((N, D), jnp.float32),
    )(partial, W, b.reshape(1, D), gamma.reshape(1, D), beta.reshape(1, D))
    return out


def kernel(x, edge_index, edge_attr, batch, W, b, gamma, beta):
    out = _run(x, edge_index[0], edge_index[1], edge_attr, W, b, gamma, beta)
    return (out, edge_index, edge_attr, batch)


# speedup vs baseline: 3.2914x; 3.2914x over previous
import functools

import jax
import jax.numpy as jnp
from jax import lax
from jax.experimental import pallas as pl
from jax.experimental.pallas import tpu as pltpu
from jax.experimental.pallas import tpu_sc as plsc

N = 10000
D = 128
E = 320000

NC = 2
NS = 16
NW = NC * NS

CHUNK = 128
EDGES_PER_TILE = -(-E // (NW * CHUNK)) * CHUNK
EPAD = EDGES_PER_TILE * NW
NCHUNK = EDGES_PER_TILE // CHUNK
ROWS_PER_TILE = -(-N // (NS * CHUNK)) * CHUNK
NPAD = ROWS_PER_TILE * NS


def _sc_body(src_hbm, dst_hbm, attr_hbm, x_hbm, out_hbm,
             src_v, dst_v, attr_v, rows_v, acc, sem):
    c = lax.axis_index("c")
    s = lax.axis_index("s")
    w = c * NS + s

    @pl.loop(0, CHUNK)
    def _zero_rows(i):
        for j in range(D // 16):
            rows_v[i, pl.ds(j * 16, 16)] = jnp.zeros((16,), jnp.float32)

    for j in range(ROWS_PER_TILE // CHUNK):
        pltpu.sync_copy(rows_v, acc.at[pl.ds(s * ROWS_PER_TILE + j * CHUNK, CHUNK)])
    plsc.subcore_barrier()

    base0 = w * EDGES_PER_TILE

    @pl.loop(0, NCHUNK)
    def _edge_chunk(g):
        base = base0 + g * CHUNK
        pltpu.sync_copy(src_hbm.at[pl.ds(base, CHUNK)], src_v)
        pltpu.sync_copy(dst_hbm.at[pl.ds(base, CHUNK)], dst_v)
        pltpu.sync_copy(attr_hbm.at[pl.ds(base, CHUNK)], attr_v.at[pl.ds(0, CHUNK)])
        pltpu.async_copy(x_hbm.at[src_v], rows_v, sem).wait()

        @pl.loop(0, CHUNK)
        def _scale(e):
            a = attr_v[pl.ds(e, 16)][0]
            for j in range(D // 16):
                sl = pl.ds(j * 16, 16)
                rows_v[e, sl] = rows_v[e, sl] * a

        pltpu.sync_copy(rows_v, acc.at[dst_v], add=True)

    plsc.subcore_barrier()

    obase = c * NPAD + s * ROWS_PER_TILE
    for j in range(ROWS_PER_TILE // CHUNK):
        pltpu.sync_copy(acc.at[pl.ds(s * ROWS_PER_TILE + j * CHUNK, CHUNK)], rows_v)
        pltpu.sync_copy(rows_v, out_hbm.at[pl.ds(obase + j * CHUNK, CHUNK)])


_sc_agg = pl.kernel(
    _sc_body,
    out_type=jax.ShapeDtypeStruct((NC * NPAD, D), jnp.float32),
    mesh=plsc.VectorSubcoreMesh(core_axis_name="c", subcore_axis_name="s"),
    scratch_types=[
        pltpu.VMEM((CHUNK,), jnp.int32),
        pltpu.VMEM((CHUNK,), jnp.int32),
        pltpu.VMEM((CHUNK + 16,), jnp.float32),
        pltpu.VMEM((CHUNK, D), jnp.float32),
        pltpu.VMEM_SHARED((NPAD, D), jnp.float32),
        pltpu.SemaphoreType.DMA,
    ],
)


def _tc_body(part_ref, w_ref, b_ref, gamma_ref, beta_ref, out_ref):
    agg = part_ref[0:N, :] + part_ref[NPAD:NPAD + N, :]
    y = jnp.dot(agg, w_ref[...], preferred_element_type=jnp.float32)
    y = y + b_ref[...]
    mean = jnp.mean(y, axis=0, keepdims=True)
    yc = y - mean
    var = jnp.mean(yc * yc, axis=0, keepdims=True)
    scale = lax.rsqrt(var + 1e-5) * gamma_ref[...]
    out_ref[...] = jnp.maximum(yc * scale + beta_ref[...], 0.0)


@jax.jit
def _run(x, src, dst, attr, W, b, gamma, beta):
    pad = EPAD - E
    src_p = jnp.pad(src, (0, pad))
    dst_p = jnp.pad(dst, (0, pad))
    attr_p = jnp.pad(attr, (0, pad))

    partial = _sc_agg(src_p, dst_p, attr_p, x)

    out = pl.pallas_call(
        _tc_body,
        out_shape=jax.ShapeDtypeStruct((N, D), jnp.float32),
    )(partial, W, b.reshape(1, D), gamma.reshape(1, D), beta.reshape(1, D))
    return out


def kernel(x, edge_index, edge_attr, batch, W, b, gamma, beta):
    out = _run(x, edge_index[0], edge_index[1], edge_attr, W, b, gamma, beta)
    return (out, edge_index, edge_attr, batch)


# speedup vs baseline: 4.0608x; 1.2338x over previous
import jax
import jax.numpy as jnp
from jax import lax
from jax.experimental import pallas as pl
from jax.experimental.pallas import tpu as pltpu
from jax.experimental.pallas import tpu_sc as plsc

N = 10000
D = 128
E = 320000

NC = 2
NS = 16
NW = NC * NS

CHUNK = 128
EDGES_PER_TILE = -(-E // (NW * CHUNK)) * CHUNK
EPAD = EDGES_PER_TILE * NW
NCHUNK = EDGES_PER_TILE // CHUNK
ROWS_PER_TILE = -(-N // (NS * CHUNK)) * CHUNK
NPAD = ROWS_PER_TILE * NS


def _sc_body(edges_hbm, x_hbm, out_hbm, edges_v, rows_v, acc, sem):
    c = lax.axis_index("c")
    s = lax.axis_index("s")
    w = c * NS + s

    @pl.loop(0, CHUNK)
    def _zero_rows(i):
        for j in range(D // 16):
            rows_v[i, pl.ds(j * 16, 16)] = jnp.zeros((16,), jnp.float32)

    for j in range(ROWS_PER_TILE // CHUNK):
        pltpu.sync_copy(rows_v, acc.at[pl.ds(s * ROWS_PER_TILE + j * CHUNK, CHUNK)])
    plsc.subcore_barrier()

    @pl.loop(0, NCHUNK)
    def _edge_chunk(g):
        pltpu.sync_copy(edges_hbm.at[w, g], edges_v)
        pltpu.async_copy(x_hbm.at[edges_v.at[0]], rows_v, sem).wait()

        @pl.loop(0, CHUNK // 16)
        def _scale_group(grp):
            av = edges_v[2, pl.ds(grp * 16, 16)]
            for l in range(16):
                a = lax.bitcast_convert_type(av[l], jnp.float32)
                e = grp * 16 + l
                for j in range(D // 16):
                    sl = pl.ds(j * 16, 16)
                    rows_v[e, sl] = rows_v[e, sl] * a

        pltpu.sync_copy(rows_v, acc.at[edges_v.at[1]], add=True)

    plsc.subcore_barrier()

    pltpu.sync_copy(
        acc.at[pl.ds(s * ROWS_PER_TILE, ROWS_PER_TILE)],
        out_hbm.at[pl.ds(c * NPAD + s * ROWS_PER_TILE, ROWS_PER_TILE)],
    )


_sc_agg = pl.kernel(
    _sc_body,
    out_type=jax.ShapeDtypeStruct((NC * NPAD, D), jnp.float32),
    mesh=plsc.VectorSubcoreMesh(core_axis_name="c", subcore_axis_name="s"),
    scratch_types=[
        pltpu.VMEM((3, CHUNK), jnp.int32),
        pltpu.VMEM((CHUNK, D), jnp.float32),
        pltpu.VMEM_SHARED((NPAD, D), jnp.float32),
        pltpu.SemaphoreType.DMA,
    ],
)


def _tc_body(part_ref, w_ref, b_ref, gamma_ref, beta_ref, out_ref):
    agg = part_ref[0:N, :] + part_ref[NPAD:NPAD + N, :]
    y = jnp.dot(agg, w_ref[...], preferred_element_type=jnp.float32)
    y = y + b_ref[...]
    mean = jnp.mean(y, axis=0, keepdims=True)
    yc = y - mean
    var = jnp.mean(yc * yc, axis=0, keepdims=True)
    scale = lax.rsqrt(var + 1e-5) * gamma_ref[...]
    out_ref[...] = jnp.maximum(yc * scale + beta_ref[...], 0.0)


@jax.jit
def _run(x, src, dst, attr, W, b, gamma, beta):
    pad = EPAD - E
    packed = jnp.stack(
        [
            jnp.pad(src, (0, pad)),
            jnp.pad(dst, (0, pad)),
            lax.bitcast_convert_type(jnp.pad(attr, (0, pad)), jnp.int32),
        ],
        axis=0,
    )
    packed = packed.reshape(3, NW, NCHUNK, CHUNK).transpose(1, 2, 0, 3)

    partial = _sc_agg(packed, x)

    out = pl.pallas_call(
        _tc_body,
        out_shape=jax.ShapeDtypeStruct((N, D), jnp.float32),
    )(partial, W, b.reshape(1, D), gamma.reshape(1, D), beta.reshape(1, D))
    return out


def kernel(x, edge_index, edge_attr, batch, W, b, gamma, beta):
    out = _run(x, edge_index[0], edge_index[1], edge_attr, W, b, gamma, beta)
    return (out, edge_index, edge_attr, batch)
